# baseline (speedup 1.0000x reference)
import functools

import jax
import jax.numpy as jnp
from jax import lax
from jax.experimental import pallas as pl
from jax.experimental.pallas import tpu as pltpu
from jax.experimental.pallas import tpu_sc as plsc

NUM_EMB = 1000
SEM_IDS_DIM = 4
EMB_DIM = 128
PAD_IDX = SEM_IDS_DIM * NUM_EMB
B, L = 4096, 200
NSEQ = B * L
NFUT = B * SEM_IDS_DIM
NROWS = NUM_EMB * SEM_IDS_DIM + 1

NC, NS = 2, 16
NW = NC * NS
ROWS_PER_W = B // NW
GRP = 8
NGRP = ROWS_PER_W // GRP
DIDX = L
NSLICE = 12
TAILC = 184
GCH = [128] * 12 + [64]
NRB = 4
FCHUNK = 128
FUT_PER_W = NFUT // NW
FUT_CHUNKS = FUT_PER_W // FCHUNK
STAGE = 256


def _body(tt, sem, msk, ttf, semf, emb, seq_out, fut_out,
          ttg0, ttg1, smg0, smg1, mkg0, mkg1, ix0, ix1,
          rw0, rw1, rw2, rw3,
          ftt0, ftt1, fsm0, fsm1, emb_sh,
          in_s0, in_s1, g_s0, g_s1, g_s2, g_s3,
          o_s0, o_s1, o_s2, o_s3, f_s0, f_s1):
    cid = lax.axis_index("c")
    sid = lax.axis_index("s")
    wid = sid * NC + cid
    ttg = (ttg0, ttg1)
    smg = (smg0, smg1)
    mkg = (mkg0, mkg1)
    idx_v = (ix0, ix1)
    rows_v = (rw0, rw1, rw2, rw3)
    ftt = (ftt0, ftt1)
    fsm = (fsm0, fsm1)
    in_sems = (in_s0, in_s1)
    g_sems = (g_s0, g_s1, g_s2, g_s3)
    o_sems = (o_s0, o_s1, o_s2, o_s3)
    f_sems = (f_s0, f_s1)

    @pl.when(sid < NS - 1)
    def _():
        for h in range(2):
            base = sid * STAGE + h * 128
            pltpu.sync_copy(emb.at[pl.ds(base, 128)],
                            rows_v[h].at[pl.ds(0, 128)])
            pltpu.sync_copy(rows_v[h].at[pl.ds(0, 128)],
                            emb_sh.at[pl.ds(base, 128)])

    @pl.when(sid == NS - 1)
    def _():
        tail = NROWS - (NS - 1) * STAGE
        tbase = (NS - 1) * STAGE
        pltpu.sync_copy(emb.at[pl.ds(tbase, 128)],
                        rows_v[0].at[pl.ds(0, 128)])
        pltpu.sync_copy(rows_v[0].at[pl.ds(0, 128)],
                        emb_sh.at[pl.ds(tbase, 128)])
        rest = rows_v[1].at[pl.ds(0, tail - 128)]
        pltpu.sync_copy(emb.at[pl.ds(tbase + 128, tail - 128)], rest)
        pltpu.sync_copy(rest, emb_sh.at[pl.ds(tbase + 128, tail - 128)])

    plsc.subcore_barrier()

    rbase = wid * ROWS_PER_W
    obase = wid * ROWS_PER_W * L

    def issue_grp(b, g):
        r0 = rbase + GRP * g
        pltpu.async_copy(tt.at[pl.ds(r0, GRP), :], ttg[b], in_sems[b])
        pltpu.async_copy(sem.at[pl.ds(r0, GRP), :], smg[b], in_sems[b])
        pltpu.async_copy(msk.at[pl.ds(r0, GRP), :], mkg[b], in_sems[b])

    def wait_grp(b):
        for ref in (ttg[b], smg[b], mkg[b]):
            pltpu.make_async_copy(tt.at[pl.ds(0, GRP), :], ref,
                                  in_sems[b]).wait()

    def compute_one(b, r, c, ic):
        t = ttg[b][r, pl.ds(c, 16)]
        s = smg[b][r, pl.ds(c, 16)]
        m = mkg[b][r, pl.ds(c, 16)]
        ids = jnp.where(m != 0, t * NUM_EMB + s, PAD_IDX)
        idx_v[b][pl.ds(DIDX * r + ic, 16)] = ids

    def compute_grp(b):
        for r in range(GRP):
            for i in range(NSLICE):
                compute_one(b, r, i * 16, i * 16)
            compute_one(b, r, TAILC, TAILC)

    def wait_out(slot, size):
        pltpu.make_async_copy(rows_v[slot].at[pl.ds(0, size)],
                              seq_out.at[pl.ds(0, size)],
                              o_sems[slot]).wait()

    issue_grp(0, 0)
    issue_grp(1, 1)

    def pair(p, carry):
        for q in range(2):
            g = 2 * p + q
            b = q
            wait_grp(b)
            compute_grp(b)

            @pl.when(p < NGRP // 2 - 1)
            def _():
                issue_grp(b, g + 2)

            for j in range(len(GCH)):
                size = GCH[j]
                slot = 2 * q + j % 2
                wsz = 64 if j == 0 else 128

                @pl.when(g >= 2)
                def _():
                    wait_out(slot, wsz)

                pltpu.async_copy(
                    emb_sh.at[idx_v[b].at[pl.ds(128 * j, size)]],
                    rows_v[slot].at[pl.ds(0, size)], g_sems[slot]).wait()
                pltpu.async_copy(
                    rows_v[slot].at[pl.ds(0, size)],
                    seq_out.at[pl.ds(obase + GRP * L * g + 128 * j, size)],
                    o_sems[slot])
        return carry

    lax.fori_loop(0, NGRP // 2, pair, 0)

    fbase = wid * FUT_PER_W

    def wait_fut_out(s):
        pltpu.make_async_copy(rows_v[s].at[pl.ds(0, FCHUNK)],
                              fut_out.at[pl.ds(0, FCHUNK)],
                              o_sems[s]).wait()

    for f in range(2):
        pltpu.async_copy(ttf.at[pl.ds(fbase + f * FCHUNK, FCHUNK)],
                         ftt[f], f_sems[f])
        pltpu.async_copy(semf.at[pl.ds(fbase + f * FCHUNK, FCHUNK)],
                         fsm[f], f_sems[f])

    for f in range(FUT_CHUNKS):
        s = f % 2
        pltpu.make_async_copy(ttf.at[pl.ds(0, FCHUNK)], ftt[s],
                              f_sems[s]).wait()
        pltpu.make_async_copy(semf.at[pl.ds(0, FCHUNK)], fsm[s],
                              f_sems[s]).wait()
        for i in range(FCHUNK // 16):
            tv = ftt[s][pl.ds(i * 16, 16)]
            sv = fsm[s][pl.ds(i * 16, 16)]
            idx_v[s][pl.ds(i * 16, 16)] = tv * NUM_EMB + sv
        if f + 2 < FUT_CHUNKS:
            pltpu.async_copy(
                ttf.at[pl.ds(fbase + (f + 2) * FCHUNK, FCHUNK)],
                ftt[s], f_sems[s])
            pltpu.async_copy(
                semf.at[pl.ds(fbase + (f + 2) * FCHUNK, FCHUNK)],
                fsm[s], f_sems[s])
        if f < 2:
            wait_out(f, 64 if f == 0 else 128)
        else:
            wait_fut_out(s)
        pltpu.async_copy(
            emb_sh.at[idx_v[s].at[pl.ds(0, FCHUNK)]],
            rows_v[s].at[pl.ds(0, FCHUNK)], g_sems[s]).wait()
        pltpu.async_copy(rows_v[s].at[pl.ds(0, FCHUNK)],
                         fut_out.at[pl.ds(fbase + f * FCHUNK, FCHUNK)],
                         o_sems[s])

    for s in range(2):
        wait_fut_out(s)
    wait_out(2, 64)
    wait_out(3, 128)


_sc_lookup = functools.partial(
    pl.kernel,
    out_type=[
        jax.ShapeDtypeStruct((NSEQ, EMB_DIM), jnp.float32),
        jax.ShapeDtypeStruct((NFUT, EMB_DIM), jnp.float32),
    ],
    mesh=plsc.VectorSubcoreMesh(core_axis_name="c", subcore_axis_name="s"),
    scratch_types=(
        [pltpu.VMEM((GRP, L), jnp.int32)] * 6
        + [pltpu.VMEM((GRP * L,), jnp.int32)] * 2
        + [pltpu.VMEM((FCHUNK, EMB_DIM), jnp.float32)] * 4
        + [pltpu.VMEM((FCHUNK,), jnp.int32)] * 4
        + [pltpu.VMEM_SHARED((NROWS, EMB_DIM), jnp.float32)]
        + [pltpu.SemaphoreType.DMA] * 12),
)(_body)


def kernel(token_type_ids, sem_ids, seq_mask, sem_ids_fut, token_type_ids_fut, emb):
    tt = token_type_ids.astype(jnp.int32)
    sm = sem_ids.astype(jnp.int32)
    mk = seq_mask.astype(jnp.int32)
    ttf = token_type_ids_fut.astype(jnp.int32).reshape(NFUT)
    smf = sem_ids_fut.astype(jnp.int32).reshape(NFUT)
    seq_flat, fut_flat = _sc_lookup(tt, sm, mk, ttf, smf, emb)
    return (seq_flat.reshape(B, L, EMB_DIM),
            fut_flat.reshape(B, SEM_IDS_DIM, EMB_DIM))


# speedup vs baseline: 1.2957x; 1.2957x over previous
import functools

import jax
import jax.numpy as jnp
from jax import lax
from jax.experimental import pallas as pl
from jax.experimental.pallas import tpu as pltpu
from jax.experimental.pallas import tpu_sc as plsc

NUM_EMB = 1000
SEM_IDS_DIM = 4
EMB_DIM = 128
PAD_IDX = SEM_IDS_DIM * NUM_EMB
B, L = 4096, 200
NSEQ = B * L
NFUT = B * SEM_IDS_DIM
NROWS = NUM_EMB * SEM_IDS_DIM + 1

NC, NS = 2, 16
NW = NC * NS
ROWS_PER_W = B // NW
GRP = 8
NGRP = ROWS_PER_W // GRP
DIDX = L
NSLICE = 12
TAILC = 184
GCH = [128] * 12 + [64]
NRB = 4
FCHUNK = 128
FUT_PER_W = NFUT // NW
FUT_CHUNKS = FUT_PER_W // FCHUNK
STAGE = 256
PADBASE = 4008
NPADROWS = 512


def _body(tt, sem, msk, ttf, semf, emb, seq_out, fut_out,
          ttg0, ttg1, smg0, smg1, mkg0, mkg1, ix0, ix1,
          rw0, rw1, rw2, rw3,
          ftt0, ftt1, fsm0, fsm1, emb_sh,
          in_s0, in_s1, g_s0, g_s1, g_s2, g_s3,
          o_s0, o_s1, o_s2, o_s3, f_s0, f_s1):
    cid = lax.axis_index("c")
    sid = lax.axis_index("s")
    wid = sid * NC + cid
    ttg = (ttg0, ttg1)
    smg = (smg0, smg1)
    mkg = (mkg0, mkg1)
    idx_v = (ix0, ix1)
    rows_v = (rw0, rw1, rw2, rw3)
    ftt = (ftt0, ftt1)
    fsm = (fsm0, fsm1)
    in_sems = (in_s0, in_s1)
    g_sems = (g_s0, g_s1, g_s2, g_s3)
    o_sems = (o_s0, o_s1, o_s2, o_s3)
    f_sems = (f_s0, f_s1)

    @pl.when(sid < NS - 1)
    def _():
        for h in range(2):
            base = sid * STAGE + h * 128
            pltpu.sync_copy(emb.at[pl.ds(base, 128)],
                            rows_v[h].at[pl.ds(0, 128)])
            pltpu.sync_copy(rows_v[h].at[pl.ds(0, 128)],
                            emb_sh.at[pl.ds(base, 128)])

    @pl.when(sid == NS - 1)
    def _():
        tail = NROWS - (NS - 1) * STAGE
        tbase = (NS - 1) * STAGE
        pltpu.sync_copy(emb.at[pl.ds(tbase, 128)],
                        rows_v[0].at[pl.ds(0, 128)])
        pltpu.sync_copy(rows_v[0].at[pl.ds(0, 128)],
                        emb_sh.at[pl.ds(tbase, 128)])
        rest = rows_v[1].at[pl.ds(0, tail - 128)]
        pltpu.sync_copy(emb.at[pl.ds(tbase + 128, tail - 128)], rest)
        pltpu.sync_copy(rest, emb_sh.at[pl.ds(tbase + 128, tail - 128)])

    @pl.when(sid == 0)
    def _():
        zeros16 = jnp.zeros((16,), jnp.float32)
        for r in range(16):
            for c in range(EMB_DIM // 16):
                rows_v[2][r, pl.ds(16 * c, 16)] = zeros16
        z16 = rows_v[2].at[pl.ds(0, 16)]
        for k in range(NPADROWS // 16):
            pltpu.sync_copy(z16, emb_sh.at[pl.ds(PADBASE + 16 * k, 16)])

    plsc.subcore_barrier()

    rbase = wid * ROWS_PER_W
    obase = wid * ROWS_PER_W * L

    def issue_grp(b, g):
        r0 = rbase + GRP * g
        pltpu.async_copy(tt.at[pl.ds(r0, GRP), :], ttg[b], in_sems[b])
        pltpu.async_copy(sem.at[pl.ds(r0, GRP), :], smg[b], in_sems[b])
        pltpu.async_copy(msk.at[pl.ds(r0, GRP), :], mkg[b], in_sems[b])

    def wait_grp(b):
        for ref in (ttg[b], smg[b], mkg[b]):
            pltpu.make_async_copy(tt.at[pl.ds(0, GRP), :], ref,
                                  in_sems[b]).wait()

    def compute_one(b, r, c, ic):
        t = ttg[b][r, pl.ds(c, 16)]
        s = smg[b][r, pl.ds(c, 16)]
        m = mkg[b][r, pl.ds(c, 16)]
        padv = PADBASE + ((DIDX * r + ic + lax.iota(jnp.int32, 16))
                          & (NPADROWS - 1))
        ids = jnp.where(m != 0, t * NUM_EMB + s, padv)
        idx_v[b][pl.ds(DIDX * r + ic, 16)] = ids

    def compute_grp(b):
        for r in range(GRP):
            for i in range(NSLICE):
                compute_one(b, r, i * 16, i * 16)
            compute_one(b, r, TAILC, TAILC)

    def wait_out(slot, size):
        pltpu.make_async_copy(rows_v[slot].at[pl.ds(0, size)],
                              seq_out.at[pl.ds(0, size)],
                              o_sems[slot]).wait()

    issue_grp(0, 0)
    issue_grp(1, 1)

    def pair(p, carry):
        for q in range(2):
            g = 2 * p + q
            b = q
            wait_grp(b)
            compute_grp(b)

            @pl.when(p < NGRP // 2 - 1)
            def _():
                issue_grp(b, g + 2)

            for j in range(len(GCH)):
                size = GCH[j]
                slot = 2 * q + j % 2
                wsz = 64 if j == 0 else 128

                @pl.when(g >= 2)
                def _():
                    wait_out(slot, wsz)

                pltpu.async_copy(
                    emb_sh.at[idx_v[b].at[pl.ds(128 * j, size)]],
                    rows_v[slot].at[pl.ds(0, size)], g_sems[slot]).wait()
                pltpu.async_copy(
                    rows_v[slot].at[pl.ds(0, size)],
                    seq_out.at[pl.ds(obase + GRP * L * g + 128 * j, size)],
                    o_sems[slot])
        return carry

    lax.fori_loop(0, NGRP // 2, pair, 0)

    fbase = wid * FUT_PER_W

    def wait_fut_out(s):
        pltpu.make_async_copy(rows_v[s].at[pl.ds(0, FCHUNK)],
                              fut_out.at[pl.ds(0, FCHUNK)],
                              o_sems[s]).wait()

    for f in range(2):
        pltpu.async_copy(ttf.at[pl.ds(fbase + f * FCHUNK, FCHUNK)],
                         ftt[f], f_sems[f])
        pltpu.async_copy(semf.at[pl.ds(fbase + f * FCHUNK, FCHUNK)],
                         fsm[f], f_sems[f])

    for f in range(FUT_CHUNKS):
        s = f % 2
        pltpu.make_async_copy(ttf.at[pl.ds(0, FCHUNK)], ftt[s],
                              f_sems[s]).wait()
        pltpu.make_async_copy(semf.at[pl.ds(0, FCHUNK)], fsm[s],
                              f_sems[s]).wait()
        for i in range(FCHUNK // 16):
            tv = ftt[s][pl.ds(i * 16, 16)]
            sv = fsm[s][pl.ds(i * 16, 16)]
            idx_v[s][pl.ds(i * 16, 16)] = tv * NUM_EMB + sv
        if f + 2 < FUT_CHUNKS:
            pltpu.async_copy(
                ttf.at[pl.ds(fbase + (f + 2) * FCHUNK, FCHUNK)],
                ftt[s], f_sems[s])
            pltpu.async_copy(
                semf.at[pl.ds(fbase + (f + 2) * FCHUNK, FCHUNK)],
                fsm[s], f_sems[s])
        if f < 2:
            wait_out(f, 64 if f == 0 else 128)
        else:
            wait_fut_out(s)
        pltpu.async_copy(
            emb_sh.at[idx_v[s].at[pl.ds(0, FCHUNK)]],
            rows_v[s].at[pl.ds(0, FCHUNK)], g_sems[s]).wait()
        pltpu.async_copy(rows_v[s].at[pl.ds(0, FCHUNK)],
                         fut_out.at[pl.ds(fbase + f * FCHUNK, FCHUNK)],
                         o_sems[s])

    for s in range(2):
        wait_fut_out(s)
    wait_out(2, 64)
    wait_out(3, 128)


_sc_lookup = functools.partial(
    pl.kernel,
    out_type=[
        jax.ShapeDtypeStruct((NSEQ, EMB_DIM), jnp.float32),
        jax.ShapeDtypeStruct((NFUT, EMB_DIM), jnp.float32),
    ],
    mesh=plsc.VectorSubcoreMesh(core_axis_name="c", subcore_axis_name="s"),
    scratch_types=(
        [pltpu.VMEM((GRP, L), jnp.int32)] * 6
        + [pltpu.VMEM((GRP * L,), jnp.int32)] * 2
        + [pltpu.VMEM((FCHUNK, EMB_DIM), jnp.float32)] * 4
        + [pltpu.VMEM((FCHUNK,), jnp.int32)] * 4
        + [pltpu.VMEM_SHARED((PADBASE + NPADROWS, EMB_DIM), jnp.float32)]
        + [pltpu.SemaphoreType.DMA] * 12),
)(_body)


def kernel(token_type_ids, sem_ids, seq_mask, sem_ids_fut, token_type_ids_fut, emb):
    tt = token_type_ids.astype(jnp.int32)
    sm = sem_ids.astype(jnp.int32)
    mk = seq_mask.astype(jnp.int32)
    ttf = token_type_ids_fut.astype(jnp.int32).reshape(NFUT)
    smf = sem_ids_fut.astype(jnp.int32).reshape(NFUT)
    seq_flat, fut_flat = _sc_lookup(tt, sm, mk, ttf, smf, emb)
    return (seq_flat.reshape(B, L, EMB_DIM),
            fut_flat.reshape(B, SEM_IDS_DIM, EMB_DIM))
